# baseline (speedup 1.0000x reference)
import functools

import jax
import jax.numpy as jnp
from jax import lax
from jax.experimental import pallas as pl
from jax.experimental.pallas import tpu as pltpu
from jax.experimental.pallas import tpu_sc as plsc

_N, _D, _EMB, _K = 10000, 32, 128, 16
_L = 16
_NW = 32
_BNS = 8
_NCH = _EMB // _L

_NS = 3840
_NT = _N - _NS
_BN_TC = 80
_RPW = _NS // _NW
_NBLK = _RPW // _BNS
_OGRP = 5
_BN_PJ = 256


def _vgather(vec, idx):
    dnums = lax.GatherDimensionNumbers(
        offset_dims=(), collapsed_slice_dims=(0,), start_index_map=(0,))
    return lax.gather(vec, idx[:, None], dnums, (1,),
                      mode=lax.GatherScatterMode.PROMISE_IN_BOUNDS)


def _sc_agg_body(msg_hbm, ec_hbm, et_hbm, out_hbm,
                 msgbuf0, msgbuf1, ecbuf, etbuf, outbuf, sem0, sem1):
    c = lax.axis_index("c")
    s = lax.axis_index("s")
    wid = s * 2 + c
    row0 = wid * _RPW

    pltpu.sync_copy(ec_hbm.at[pl.ds(row0 * _K, _RPW * _K)], ecbuf)
    pltpu.sync_copy(et_hbm.at[pl.ds(row0 * _D, _RPW * _D)], etbuf)

    msgbufs = (msgbuf0, msgbuf1)
    sems = (sem0, sem1)

    def _dma(blk, b):
        return pltpu.make_async_copy(
            msg_hbm.at[pl.ds(row0 + blk * _BNS, _BNS)], msgbufs[b], sems[b])

    _dma(0, 0).start()
    _dma(1, 1).start()

    def process_block(blk, b):
            _dma(blk, b).wait()
            mb = msgbufs[b]
            for i in range(_BNS):
                node = blk * _BNS + i
                ec = ecbuf[pl.ds(node * _K, _K)]
                et0 = etbuf[pl.ds(node * _D, _L)]
                et1 = etbuf[pl.ds(node * _D + _L, _L)]
                w0v = 1.0 / _vgather(ec, et0)
                w1v = 1.0 / _vgather(ec, et1)

                def half_body(wv_src, half, i=i, mb=mb):
                    def body(k, acc, wv_src=wv_src, half=half, i=i, mb=mb):
                        nacc = list(acc)
                        for u in range(4):
                            d = k * 4 + u
                            wv = _vgather(wv_src, jnp.full((_L,), d, jnp.int32))
                            for ch in range(_NCH):
                                m = mb[i, half * _L + d, pl.ds(ch * _L, _L)]
                                nacc[ch] = nacc[ch] + wv * m
                                nacc[_NCH + ch] = nacc[_NCH + ch] + m
                        return tuple(nacc)
                    return body

                zeros = (jnp.zeros((_L,), jnp.float32),) * (2 * _NCH)
                acc = lax.fori_loop(0, _L // 4, half_body(w0v, 0), zeros)
                acc = lax.fori_loop(0, _L // 4, half_body(w1v, 1), acc)

                orow = (blk % _OGRP) * _BNS + i
                for ch in range(_NCH):
                    outbuf[orow, pl.ds(ch * _L, _L)] = acc[ch]
                    outbuf[orow, pl.ds(_EMB + ch * _L, _L)] = acc[_NCH + ch]

            @pl.when(blk + 2 < _NBLK)
            def _issue(blk=blk, b=b):
                _dma(blk + 2, b).start()

            @pl.when(blk % _OGRP == _OGRP - 1)
            def _store(blk=blk):
                r0 = row0 + (blk - (_OGRP - 1)) * _BNS
                pltpu.sync_copy(outbuf, out_hbm.at[pl.ds(r0, _OGRP * _BNS)])

    def pair_body(p, carry):
        for b in range(2):
            process_block(2 * p + b, b)
        return carry

    lax.fori_loop(0, _NBLK // 2, pair_body, 0)
    if _NBLK % 2:
        process_block(jnp.int32(_NBLK - 1), 0)


def _sc_agg(msg, ec0, e_type):
    mesh = plsc.VectorSubcoreMesh(core_axis_name="c", subcore_axis_name="s")
    f = functools.partial(
        pl.kernel,
        mesh=mesh,
        out_type=jax.ShapeDtypeStruct((_NS, 2 * _EMB), jnp.float32),
        compiler_params=pltpu.CompilerParams(use_tc_tiling_on_sc=True),
        scratch_types=[
            pltpu.VMEM((_BNS, _D, _EMB), jnp.float32),
            pltpu.VMEM((_BNS, _D, _EMB), jnp.float32),
            pltpu.VMEM((_RPW * _K,), jnp.float32),
            pltpu.VMEM((_RPW * _D,), jnp.int32),
            pltpu.VMEM((_OGRP * _BNS, 2 * _EMB), jnp.float32),
            pltpu.SemaphoreType.DMA,
            pltpu.SemaphoreType.DMA,
        ],
    )(_sc_agg_body)
    return f(msg, ec0, e_type)


def _tc_body(ec0_ref, et_ref, msg_ref, w1t_ref, w2t_ref, b_ref, out_ref):
    ec0 = ec0_ref[...]
    et = et_ref[...]
    e_total = jnp.sum(ec0, axis=1, keepdims=True)
    gathered = jnp.zeros(et.shape, jnp.float32)
    for k in range(_K):
        gathered = gathered + jnp.where(et == k, ec0[:, k:k + 1], 0.0)
    w = 1.0 / gathered
    msg = msg_ref[...]
    nei = jnp.sum(msg * w[:, :, None], axis=1)
    norm = jnp.sum(msg, axis=1) / e_total
    out1 = jnp.dot(nei, w1t_ref[...], preferred_element_type=jnp.float32)
    out2 = jnp.dot(norm, w2t_ref[...], preferred_element_type=jnp.float32)
    out_ref[...] = jnp.concatenate([out1, out2], axis=1) + b_ref[...]


_OFF = _NS // _BN_TC


def _tc_fused(ec0, e_type, msg, w1t, w2t, b):
    return pl.pallas_call(
        _tc_body,
        grid=(_NT // _BN_TC,),
        in_specs=[
            pl.BlockSpec((_BN_TC, _K), lambda i: (i + _OFF, 0)),
            pl.BlockSpec((_BN_TC, _D), lambda i: (i + _OFF, 0)),
            pl.BlockSpec((_BN_TC, _D, _EMB), lambda i: (i + _OFF, 0, 0)),
            pl.BlockSpec((_EMB, _EMB // 2), lambda i: (0, 0)),
            pl.BlockSpec((_EMB, _EMB // 2), lambda i: (0, 0)),
            pl.BlockSpec((1, _EMB), lambda i: (0, 0)),
        ],
        out_specs=pl.BlockSpec((_BN_TC, _EMB), lambda i: (i + _OFF, 0)),
        out_shape=jax.ShapeDtypeStruct((_N, _EMB), jnp.float32),
    )(ec0, e_type, msg, w1t, w2t, b)


def _proj_body(p_ref, ec0_ref, wc_ref, b_ref, full_ref, o_ref):
    del full_ref
    raw = jnp.dot(p_ref[...], wc_ref[...], preferred_element_type=jnp.float32)
    e_total = jnp.sum(ec0_ref[...], axis=1, keepdims=True)
    half = _EMB // 2
    scale = jnp.concatenate(
        [jnp.ones(raw[:, :half].shape, jnp.float32),
         jnp.broadcast_to(1.0 / e_total, raw[:, half:].shape)], axis=1)
    o_ref[...] = raw * scale + b_ref[...]


def _proj(packed, ec0, Wc, b, full):
    return pl.pallas_call(
        _proj_body,
        grid=(_NS // _BN_PJ,),
        in_specs=[
            pl.BlockSpec((_BN_PJ, 2 * _EMB), lambda i: (i, 0)),
            pl.BlockSpec((_BN_PJ, _K), lambda i: (i, 0)),
            pl.BlockSpec((2 * _EMB, _EMB), lambda i: (0, 0)),
            pl.BlockSpec((1, _EMB), lambda i: (0, 0)),
            pl.BlockSpec(memory_space=pl.ANY),
        ],
        out_specs=pl.BlockSpec((_BN_PJ, _EMB), lambda i: (i, 0)),
        out_shape=jax.ShapeDtypeStruct((_N, _EMB), jnp.float32),
        input_output_aliases={4: 0},
    )(packed, ec0, Wc, b, full)


def kernel(curr_emb, msg, e_count, W1, b1, W2, b2, e_type):
    del curr_emb
    ec0 = e_count[:, 0, :]

    w1t = W1.T
    w2t = W2.T
    half = _EMB // 2
    b = jnp.concatenate([b1, b2])[None, :]
    Wc = jnp.zeros((2 * _EMB, _EMB), jnp.float32)
    Wc = Wc.at[0:_EMB, 0:half].set(w1t)
    Wc = Wc.at[_EMB:2 * _EMB, half:_EMB].set(w2t)

    ecf = ec0[:_NS].reshape(_NS * _K)
    etf = e_type[:_NS].reshape(_NS * _D)
    packed = _sc_agg(msg, ecf, etf)
    full = _tc_fused(ec0, e_type, msg, w1t, w2t, b)
    return _proj(packed, ec0, Wc, b, full)


# speedup vs baseline: 1.3694x; 1.3694x over previous
import functools

import jax
import jax.numpy as jnp
from jax import lax
from jax.experimental import pallas as pl
from jax.experimental.pallas import tpu as pltpu
from jax.experimental.pallas import tpu_sc as plsc

_N, _D, _EMB, _K = 10000, 32, 128, 16
_L = 16
_NW = 32
_BNS = 8
_NCH = _EMB // _L

_NS = 4608
_NT = _N - _NS
_BN_TC = 512
_RPW = _NS // _NW
_NBLK = _RPW // _BNS
_OGRP = 6
_BN_PJ = 768


def _vgather(vec, idx):
    dnums = lax.GatherDimensionNumbers(
        offset_dims=(), collapsed_slice_dims=(0,), start_index_map=(0,))
    return lax.gather(vec, idx[:, None], dnums, (1,),
                      mode=lax.GatherScatterMode.PROMISE_IN_BOUNDS)


def _sc_agg_body(msg_hbm, ec_hbm, et_hbm, out_hbm,
                 msgbuf0, msgbuf1, ecbuf, etbuf, outbuf, sem0, sem1):
    c = lax.axis_index("c")
    s = lax.axis_index("s")
    wid = s * 2 + c
    row0 = wid * _RPW

    pltpu.sync_copy(ec_hbm.at[pl.ds(row0 * _K, _RPW * _K)], ecbuf)
    pltpu.sync_copy(et_hbm.at[pl.ds(row0 * _D, _RPW * _D)], etbuf)

    msgbufs = (msgbuf0, msgbuf1)
    sems = (sem0, sem1)

    def _dma(blk, b):
        return pltpu.make_async_copy(
            msg_hbm.at[pl.ds(row0 + blk * _BNS, _BNS)], msgbufs[b], sems[b])

    _dma(0, 0).start()
    _dma(1, 1).start()

    def process_block(blk, b):
            _dma(blk, b).wait()
            mb = msgbufs[b]
            for i in range(_BNS):
                node = blk * _BNS + i
                ec = ecbuf[pl.ds(node * _K, _K)]
                et0 = etbuf[pl.ds(node * _D, _L)]
                et1 = etbuf[pl.ds(node * _D + _L, _L)]
                w0v = 1.0 / _vgather(ec, et0)
                w1v = 1.0 / _vgather(ec, et1)

                def half_body(wv_src, half, i=i, mb=mb):
                    def body(k, acc, wv_src=wv_src, half=half, i=i, mb=mb):
                        nacc = list(acc)
                        for u in range(4):
                            d = k * 4 + u
                            wv = _vgather(wv_src, jnp.full((_L,), d, jnp.int32))
                            for ch in range(_NCH):
                                m = mb[i, half * _L + d, pl.ds(ch * _L, _L)]
                                nacc[ch] = nacc[ch] + wv * m
                                nacc[_NCH + ch] = nacc[_NCH + ch] + m
                        return tuple(nacc)
                    return body

                zeros = (jnp.zeros((_L,), jnp.float32),) * (2 * _NCH)
                acc = lax.fori_loop(0, _L // 4, half_body(w0v, 0), zeros)
                acc = lax.fori_loop(0, _L // 4, half_body(w1v, 1), acc)

                orow = (blk % _OGRP) * _BNS + i
                for ch in range(_NCH):
                    outbuf[orow, pl.ds(ch * _L, _L)] = acc[ch]
                    outbuf[orow, pl.ds(_EMB + ch * _L, _L)] = acc[_NCH + ch]

            @pl.when(blk + 2 < _NBLK)
            def _issue(blk=blk, b=b):
                _dma(blk + 2, b).start()

            @pl.when(blk % _OGRP == _OGRP - 1)
            def _store(blk=blk):
                r0 = row0 + (blk - (_OGRP - 1)) * _BNS
                pltpu.sync_copy(outbuf, out_hbm.at[pl.ds(r0, _OGRP * _BNS)])

    def pair_body(p, carry):
        for b in range(2):
            process_block(2 * p + b, b)
        return carry

    lax.fori_loop(0, _NBLK // 2, pair_body, 0)
    if _NBLK % 2:
        process_block(jnp.int32(_NBLK - 1), 0)


def _sc_agg(msg, ec0, e_type):
    mesh = plsc.VectorSubcoreMesh(core_axis_name="c", subcore_axis_name="s")
    f = functools.partial(
        pl.kernel,
        mesh=mesh,
        out_type=jax.ShapeDtypeStruct((_NS, 2 * _EMB), jnp.float32),
        compiler_params=pltpu.CompilerParams(use_tc_tiling_on_sc=True),
        scratch_types=[
            pltpu.VMEM((_BNS, _D, _EMB), jnp.float32),
            pltpu.VMEM((_BNS, _D, _EMB), jnp.float32),
            pltpu.VMEM((_RPW * _K,), jnp.float32),
            pltpu.VMEM((_RPW * _D,), jnp.int32),
            pltpu.VMEM((_OGRP * _BNS, 2 * _EMB), jnp.float32),
            pltpu.SemaphoreType.DMA,
            pltpu.SemaphoreType.DMA,
        ],
    )(_sc_agg_body)
    return f(msg, ec0, e_type)


def _tc_body(ec0_ref, et_ref, msg_ref, w1t_ref, w2t_ref, b_ref, out_ref):
    ec0 = ec0_ref[...]
    et = et_ref[...]
    e_total = jnp.sum(ec0, axis=1, keepdims=True)
    gathered = jnp.zeros(et.shape, jnp.float32)
    for k in range(_K):
        gathered = gathered + jnp.where(et == k, ec0[:, k:k + 1], 0.0)
    w = 1.0 / gathered
    msg = msg_ref[...]
    nei = jnp.sum(msg * w[:, :, None], axis=1)
    norm = jnp.sum(msg, axis=1) / e_total
    out1 = jnp.dot(nei, w1t_ref[...], preferred_element_type=jnp.float32)
    out2 = jnp.dot(norm, w2t_ref[...], preferred_element_type=jnp.float32)
    out_ref[...] = jnp.concatenate([out1, out2], axis=1) + b_ref[...]


_OFF = _NS // _BN_TC


def _tc_fused(ec0, e_type, msg, w1t, w2t, b):
    return pl.pallas_call(
        _tc_body,
        grid=(-(-_NT // _BN_TC),),
        in_specs=[
            pl.BlockSpec((_BN_TC, _K), lambda i: (i + _OFF, 0)),
            pl.BlockSpec((_BN_TC, _D), lambda i: (i + _OFF, 0)),
            pl.BlockSpec((_BN_TC, _D, _EMB), lambda i: (i + _OFF, 0, 0)),
            pl.BlockSpec((_EMB, _EMB // 2), lambda i: (0, 0)),
            pl.BlockSpec((_EMB, _EMB // 2), lambda i: (0, 0)),
            pl.BlockSpec((1, _EMB), lambda i: (0, 0)),
        ],
        out_specs=pl.BlockSpec((_BN_TC, _EMB), lambda i: (i + _OFF, 0)),
        out_shape=jax.ShapeDtypeStruct((_N, _EMB), jnp.float32),
    )(ec0, e_type, msg, w1t, w2t, b)


def _proj_body(p_ref, ec0_ref, wc_ref, b_ref, full_ref, o_ref):
    del full_ref
    raw = jnp.dot(p_ref[...], wc_ref[...], preferred_element_type=jnp.float32)
    e_total = jnp.sum(ec0_ref[...], axis=1, keepdims=True)
    half = _EMB // 2
    scale = jnp.concatenate(
        [jnp.ones(raw[:, :half].shape, jnp.float32),
         jnp.broadcast_to(1.0 / e_total, raw[:, half:].shape)], axis=1)
    o_ref[...] = raw * scale + b_ref[...]


def _proj(packed, ec0, Wc, b, full):
    return pl.pallas_call(
        _proj_body,
        grid=(_NS // _BN_PJ,),
        in_specs=[
            pl.BlockSpec((_BN_PJ, 2 * _EMB), lambda i: (i, 0)),
            pl.BlockSpec((_BN_PJ, _K), lambda i: (i, 0)),
            pl.BlockSpec((2 * _EMB, _EMB), lambda i: (0, 0)),
            pl.BlockSpec((1, _EMB), lambda i: (0, 0)),
            pl.BlockSpec(memory_space=pl.ANY),
        ],
        out_specs=pl.BlockSpec((_BN_PJ, _EMB), lambda i: (i, 0)),
        out_shape=jax.ShapeDtypeStruct((_N, _EMB), jnp.float32),
        input_output_aliases={4: 0},
    )(packed, ec0, Wc, b, full)


def kernel(curr_emb, msg, e_count, W1, b1, W2, b2, e_type):
    del curr_emb
    ec0 = e_count[:, 0, :]

    w1t = W1.T
    w2t = W2.T
    half = _EMB // 2
    b = jnp.concatenate([b1, b2])[None, :]
    Wc = jnp.zeros((2 * _EMB, _EMB), jnp.float32)
    Wc = Wc.at[0:_EMB, 0:half].set(w1t)
    Wc = Wc.at[_EMB:2 * _EMB, half:_EMB].set(w2t)

    ecf = ec0[:_NS].reshape(_NS * _K)
    etf = e_type[:_NS].reshape(_NS * _D)
    packed = _sc_agg(msg, ecf, etf)
    full = _tc_fused(ec0, e_type, msg, w1t, w2t, b)
    return _proj(packed, ec0, Wc, b, full)


# speedup vs baseline: 1.5193x; 1.1094x over previous
import functools

import jax
import jax.numpy as jnp
from jax import lax
from jax.experimental import pallas as pl
from jax.experimental.pallas import tpu as pltpu
from jax.experimental.pallas import tpu_sc as plsc

_N, _D, _EMB, _K = 10000, 32, 128, 16
_L = 16
_NW = 32
_BNS = 8
_NCH = _EMB // _L

_NS = 4608
_NT = _N - _NS
_BN_TC = 512
_RPW = _NS // _NW
_NBLK = _RPW // _BNS
_OGRP = 6
_BN_PJ = 768


def _vgather(vec, idx):
    dnums = lax.GatherDimensionNumbers(
        offset_dims=(), collapsed_slice_dims=(0,), start_index_map=(0,))
    return lax.gather(vec, idx[:, None], dnums, (1,),
                      mode=lax.GatherScatterMode.PROMISE_IN_BOUNDS)


def _sc_agg_body(msg_hbm, ec_hbm, et_hbm, out_hbm,
                 msgbuf0, msgbuf1, ecbuf, etbuf, outbuf, sem0, sem1):
    c = lax.axis_index("c")
    s = lax.axis_index("s")
    wid = s * 2 + c
    row0 = wid * _RPW

    pltpu.sync_copy(ec_hbm.at[pl.ds(row0 * _K, _RPW * _K)], ecbuf)
    pltpu.sync_copy(et_hbm.at[pl.ds(row0 * _D, _RPW * _D)], etbuf)

    msgbufs = (msgbuf0, msgbuf1)
    sems = (sem0, sem1)

    def _dma(blk, b):
        return pltpu.make_async_copy(
            msg_hbm.at[pl.ds(row0 + blk * _BNS, _BNS)], msgbufs[b], sems[b])

    _dma(0, 0).start()
    _dma(1, 1).start()

    def process_block(blk, b):
            _dma(blk, b).wait()
            mb = msgbufs[b]
            for i in range(_BNS):
                node = blk * _BNS + i
                ec = ecbuf[pl.ds(node * _K, _K)]
                et0 = etbuf[pl.ds(node * _D, _L)]
                et1 = etbuf[pl.ds(node * _D + _L, _L)]
                w0v = 1.0 / _vgather(ec, et0)
                w1v = 1.0 / _vgather(ec, et1)

                def half_body(wv_src, half, i=i, mb=mb):
                    def body(k, acc, wv_src=wv_src, half=half, i=i, mb=mb):
                        nacc = list(acc)
                        for u in range(4):
                            d = k * 4 + u
                            wv = _vgather(wv_src, jnp.full((_L,), d, jnp.int32))
                            for ch in range(_NCH):
                                m = mb[i, half * _L + d, pl.ds(ch * _L, _L)]
                                nacc[ch] = nacc[ch] + wv * m
                                nacc[_NCH + ch] = nacc[_NCH + ch] + m
                        return tuple(nacc)
                    return body

                zeros = (jnp.zeros((_L,), jnp.float32),) * (2 * _NCH)
                acc = lax.fori_loop(0, _L // 4, half_body(w0v, 0), zeros)
                acc = lax.fori_loop(0, _L // 4, half_body(w1v, 1), acc)

                orow = (blk % _OGRP) * _BNS + i
                for ch in range(_NCH):
                    outbuf[orow, pl.ds(ch * _L, _L)] = acc[ch]
                    outbuf[orow, pl.ds(_EMB + ch * _L, _L)] = acc[_NCH + ch]

            @pl.when(blk + 2 < _NBLK)
            def _issue(blk=blk, b=b):
                _dma(blk + 2, b).start()

            @pl.when(blk % _OGRP == _OGRP - 1)
            def _store(blk=blk):
                r0 = row0 + (blk - (_OGRP - 1)) * _BNS
                pltpu.sync_copy(outbuf, out_hbm.at[pl.ds(r0, _OGRP * _BNS)])

    def pair_body(p, carry):
        for b in range(2):
            process_block(2 * p + b, b)
        return carry

    lax.fori_loop(0, _NBLK // 2, pair_body, 0)
    if _NBLK % 2:
        process_block(jnp.int32(_NBLK - 1), 0)


def _sc_agg(msg, ec0, e_type):
    mesh = plsc.VectorSubcoreMesh(core_axis_name="c", subcore_axis_name="s")
    f = functools.partial(
        pl.kernel,
        mesh=mesh,
        out_type=jax.ShapeDtypeStruct((_NS, 2 * _EMB), jnp.float32),
        compiler_params=pltpu.CompilerParams(use_tc_tiling_on_sc=True),
        scratch_types=[
            pltpu.VMEM((_BNS, _D, _EMB), jnp.float32),
            pltpu.VMEM((_BNS, _D, _EMB), jnp.float32),
            pltpu.VMEM((_RPW * _K,), jnp.float32),
            pltpu.VMEM((_RPW * _D,), jnp.int32),
            pltpu.VMEM((_OGRP * _BNS, 2 * _EMB), jnp.float32),
            pltpu.SemaphoreType.DMA,
            pltpu.SemaphoreType.DMA,
        ],
    )(_sc_agg_body)
    return f(msg, ec0, e_type)


def _tc_body(ect_ref, ett_ref, msg_ref, w1t_ref, w2t_ref, b_ref, out_ref):
    ect = ect_ref[...]
    ett = ett_ref[...]
    e_total_t = jnp.sum(ect, axis=0, keepdims=True)
    gat_t = jnp.zeros(ett.shape, jnp.float32)
    for k in range(_K):
        gat_t = gat_t + jnp.where(ett == k, ect[k:k + 1, :], 0.0)
    wt = 1.0 / gat_t
    cat = jnp.concatenate(
        [wt, e_total_t, jnp.zeros((7, wt.shape[1]), jnp.float32)], axis=0)
    cat_t = jnp.transpose(cat)
    w = cat_t[:, :_D]
    e_total = cat_t[:, _D:_D + 1]
    msg = msg_ref[...]
    nei = jnp.sum(msg * w[:, :, None], axis=1)
    norm = jnp.sum(msg, axis=1) / e_total
    out1 = jnp.dot(nei, w1t_ref[...], preferred_element_type=jnp.float32)
    out2 = jnp.dot(norm, w2t_ref[...], preferred_element_type=jnp.float32)
    out_ref[...] = jnp.concatenate([out1, out2], axis=1) + b_ref[...]


_OFF = _NS // _BN_TC


def _tc_fused(ect, ett, msg, w1t, w2t, b):
    return pl.pallas_call(
        _tc_body,
        grid=(-(-_NT // _BN_TC),),
        in_specs=[
            pl.BlockSpec((_K, _BN_TC), lambda i: (0, i + _OFF)),
            pl.BlockSpec((_D, _BN_TC), lambda i: (0, i + _OFF)),
            pl.BlockSpec((_BN_TC, _D, _EMB), lambda i: (i + _OFF, 0, 0)),
            pl.BlockSpec((_EMB, _EMB // 2), lambda i: (0, 0)),
            pl.BlockSpec((_EMB, _EMB // 2), lambda i: (0, 0)),
            pl.BlockSpec((1, _EMB), lambda i: (0, 0)),
        ],
        out_specs=pl.BlockSpec((_BN_TC, _EMB), lambda i: (i + _OFF, 0)),
        out_shape=jax.ShapeDtypeStruct((_N, _EMB), jnp.float32),
    )(ect, ett, msg, w1t, w2t, b)


def _proj_body(p_ref, ect_ref, wc_ref, b_ref, full_ref, o_ref):
    del full_ref
    raw = jnp.dot(p_ref[...], wc_ref[...], preferred_element_type=jnp.float32)
    e_total_t = jnp.sum(ect_ref[...], axis=0, keepdims=True)
    cat = jnp.concatenate(
        [e_total_t, jnp.zeros((7, e_total_t.shape[1]), jnp.float32)], axis=0)
    e_total = jnp.transpose(cat)[:, 0:1]
    half = _EMB // 2
    scale = jnp.concatenate(
        [jnp.ones(raw[:, :half].shape, jnp.float32),
         jnp.broadcast_to(1.0 / e_total, raw[:, half:].shape)], axis=1)
    o_ref[...] = raw * scale + b_ref[...]


def _proj(packed, ect, Wc, b, full):
    return pl.pallas_call(
        _proj_body,
        grid=(_NS // _BN_PJ,),
        in_specs=[
            pl.BlockSpec((_BN_PJ, 2 * _EMB), lambda i: (i, 0)),
            pl.BlockSpec((_K, _BN_PJ), lambda i: (0, i)),
            pl.BlockSpec((2 * _EMB, _EMB), lambda i: (0, 0)),
            pl.BlockSpec((1, _EMB), lambda i: (0, 0)),
            pl.BlockSpec(memory_space=pl.ANY),
        ],
        out_specs=pl.BlockSpec((_BN_PJ, _EMB), lambda i: (i, 0)),
        out_shape=jax.ShapeDtypeStruct((_N, _EMB), jnp.float32),
        input_output_aliases={4: 0},
    )(packed, ect, Wc, b, full)


def kernel(curr_emb, msg, e_count, W1, b1, W2, b2, e_type):
    del curr_emb
    ec0 = e_count[:, 0, :]
    ect = ec0.T
    ett = e_type.T

    w1t = W1.T
    w2t = W2.T
    half = _EMB // 2
    b = jnp.concatenate([b1, b2])[None, :]
    Wc = jnp.zeros((2 * _EMB, _EMB), jnp.float32)
    Wc = Wc.at[0:_EMB, 0:half].set(w1t)
    Wc = Wc.at[_EMB:2 * _EMB, half:_EMB].set(w2t)

    ecf = ec0[:_NS].reshape(_NS * _K)
    etf = e_type[:_NS].reshape(_NS * _D)
    packed = _sc_agg(msg, ecf, etf)
    full = _tc_fused(ect, ett, msg, w1t, w2t, b)
    return _proj(packed, ect, Wc, b, full)
